# baseline (speedup 1.0000x reference)
import functools

import jax
import jax.numpy as jnp
from jax import lax
from jax.experimental import pallas as pl
from jax.experimental.pallas import tpu as pltpu
from jax.experimental.pallas import tpu_sc as plsc

_NC = 2
_NS = 16
_NW = _NC * _NS

_CK = 80


def _edge_pass(h, e, src, dst):
    n, d = h.shape
    ecount = src.shape[0]
    assert e.shape == (ecount, d) and e.dtype == jnp.float32
    assert ecount % _NW == 0
    epw = ecount // _NW
    assert epw % _CK == 0
    nchunk = epw // _CK
    assert nchunk % 2 == 1 and nchunk >= 5
    assert n % 8 == 0
    rows_a = (n // 8 // _NS) * 8
    rows_last = n - (_NS - 1) * rows_a
    zr = _CK
    assert (rows_a % zr) % 8 == 0 and (rows_last - rows_a) % 8 == 0
    assert rows_last - rows_a <= zr

    mesh = plsc.VectorSubcoreMesh(core_axis_name="c", subcore_axis_name="s",
                                  num_cores=_NC, num_subcores=_NS)

    @functools.partial(
        pl.kernel,
        out_type=jax.ShapeDtypeStruct((2 * n, d), jnp.float32),
        mesh=mesh,
        scratch_types=[
            [pltpu.VMEM((_CK,), jnp.int32) for _ in range(2)],
            [pltpu.VMEM((_CK,), jnp.int32) for _ in range(2)],
            [pltpu.VMEM((_CK, d), jnp.float32) for _ in range(2)],
            [pltpu.VMEM((_CK, d), jnp.float32) for _ in range(2)],
            pltpu.VMEM_SHARED((n, d), jnp.float32),
            [pltpu.SemaphoreType.DMA for _ in range(2)],
            [pltpu.SemaphoreType.DMA for _ in range(2)],
        ],
    )
    def body(h_hbm, e_hbm, src_hbm, dst_hbm, out_hbm,
             sidx, didx, rows, ebuf, agg, psem, gsem):
        core = lax.axis_index("c")
        sid = lax.axis_index("s")
        wid = sid * _NC + core
        base = wid * epw

        nvec = d // 16

        @plsc.parallel_loop(0, zr, 1, unroll=4)
        def _(r):
            for j in range(nvec):
                rows[0][r, pl.ds(j * 16, 16)] = jnp.zeros((16,), jnp.float32)

        base_r = sid * rows_a
        for k in range(rows_a // zr):
            pltpu.sync_copy(rows[0].at[pl.ds(0, zr)],
                            agg.at[pl.ds(base_r + k * zr, zr)])
        rem = rows_a - (rows_a // zr) * zr
        if rem:
            pltpu.sync_copy(rows[0].at[pl.ds(0, rem)],
                            agg.at[pl.ds(base_r + rows_a - rem, rem)])
        extra = rows_last - rows_a

        @pl.when(sid == _NS - 1)
        def _():
            pltpu.sync_copy(rows[0].at[pl.ds(0, extra)],
                            agg.at[pl.ds(base_r + rows_a, extra)])

        plsc.subcore_barrier()

        def prefetch_copies(c, b):
            off = base + c * _CK
            return [
                (src_hbm.at[pl.ds(off, _CK)], sidx[b]),
                (dst_hbm.at[pl.ds(off, _CK)], didx[b]),
                (e_hbm.at[pl.ds(off, _CK)], ebuf[b]),
            ]

        def fire_prefetch(c, b):
            for s, t in prefetch_copies(c, b):
                pltpu.async_copy(s, t, psem[b])

        def wait_prefetch(c, b):
            for s, t in prefetch_copies(c, b):
                pltpu.make_async_copy(s, t, psem[b]).wait()

        def gather_copies(b):
            return [(h_hbm.at[sidx[b]], rows[b])]

        def fire_gather(b):
            for s, t in gather_copies(b):
                pltpu.async_copy(s, t, gsem[b])

        def wait_gather(b):
            for s, t in gather_copies(b):
                pltpu.make_async_copy(s, t, gsem[b]).wait()

        def compute(b):
            @plsc.parallel_loop(0, _CK, 1, unroll=4)
            def _(r):
                for j in range(nvec):
                    sl = pl.ds(j * 16, 16)
                    rows[b][r, sl] = jnp.maximum(
                        rows[b][r, sl] + ebuf[b][r, sl], 0.0)

        def scatter(b):
            pltpu.sync_copy(rows[b], agg.at[didx[b]], add=True)

        def half(c, b):
            bn = 1 - b
            wait_prefetch(c + 1, bn)
            fire_gather(bn)
            wait_gather(b)
            compute(b)
            scatter(b)
            fire_prefetch(c + 2, b)

        fire_prefetch(0, 0)
        wait_prefetch(0, 0)
        fire_gather(0)
        fire_prefetch(1, 1)

        def pair(k, _):
            c = 2 * k
            half(c, 0)
            half(c + 1, 1)
            return 0

        lax.fori_loop(0, (nchunk - 3) // 2, pair, 0)

        half(nchunk - 3, 0)
        wait_prefetch(nchunk - 1, 0)
        fire_gather(0)
        wait_gather(1)
        compute(1)
        scatter(1)
        wait_gather(0)
        compute(0)
        scatter(0)

        plsc.subcore_barrier()

        @pl.when(sid < _NS - 1)
        def _():
            pltpu.sync_copy(agg.at[pl.ds(base_r, rows_a)],
                            out_hbm.at[pl.ds(core * n + base_r, rows_a)])

        @pl.when(sid == _NS - 1)
        def _():
            pltpu.sync_copy(agg.at[pl.ds(base_r, rows_last)],
                            out_hbm.at[pl.ds(core * n + base_r, rows_last)])

    return body(h, e, src, dst)


def _edge_embed1(edge_attr, edge_index, We, b):
    ecount, de = edge_attr.shape
    d1 = We.shape[1]
    be = 3200
    assert ecount % be == 0
    grid = ecount // be

    def body(ea_ref, ei_ref, w_ref, b_ref, o_ref, src_ref, dst_ref):
        ea = ea_ref[...]
        o_ref[...] = jnp.dot(ea, w_ref[...],
                             preferred_element_type=jnp.float32) + b_ref[...]

        @pl.when(pl.program_id(0) == 0)
        def _():
            src_ref[...] = ei_ref[0, :]
            dst_ref[...] = ei_ref[1, :]

    return pl.pallas_call(
        body,
        grid=(grid,),
        in_specs=[
            pl.BlockSpec((be, de), lambda i: (i, 0)),
            pl.BlockSpec((2, ecount), lambda i: (0, 0)),
            pl.BlockSpec((de, d1), lambda i: (0, 0)),
            pl.BlockSpec((1, d1), lambda i: (0, 0)),
        ],
        out_specs=[
            pl.BlockSpec((be, d1), lambda i: (i, 0)),
            pl.BlockSpec((ecount,), lambda i: (0,)),
            pl.BlockSpec((ecount,), lambda i: (0,)),
        ],
        out_shape=[
            jax.ShapeDtypeStruct((ecount, d1), jnp.float32),
            jax.ShapeDtypeStruct((ecount,), jnp.int32),
            jax.ShapeDtypeStruct((ecount,), jnp.int32),
        ],
    )(edge_attr, edge_index, We, b.reshape(1, d1))


def _edge_embed2(edge_attr, We, b):
    ecount, de = edge_attr.shape
    d1 = We.shape[1]
    be = 3200
    assert ecount % be == 0
    grid = ecount // be

    def body(ea_ref, w_ref, b_ref, o_ref):
        o_ref[...] = jnp.dot(ea_ref[...], w_ref[...],
                             preferred_element_type=jnp.float32) + b_ref[...]

    return pl.pallas_call(
        body,
        grid=(grid,),
        in_specs=[
            pl.BlockSpec((be, de), lambda i: (i, 0)),
            pl.BlockSpec((de, d1), lambda i: (0, 0)),
            pl.BlockSpec((1, d1), lambda i: (0, 0)),
        ],
        out_specs=pl.BlockSpec((be, d1), lambda i: (i, 0)),
        out_shape=jax.ShapeDtypeStruct((ecount, d1), jnp.float32),
    )(edge_attr, We, b.reshape(1, d1))


def _pallas_copy(x):
    n, d = x.shape
    bn = 400
    assert n % bn == 0

    def body(i_ref, o_ref):
        o_ref[...] = i_ref[...]

    return pl.pallas_call(
        body,
        grid=(n // bn,),
        in_specs=[pl.BlockSpec((bn, d), lambda i: (i, 0))],
        out_specs=pl.BlockSpec((bn, d), lambda i: (i, 0)),
        out_shape=jax.ShapeDtypeStruct((n, d), jnp.float32),
    )(x)


def _node_update(hin, agg2, Wa, ba, Wb, bb):
    n, d = hin.shape
    h1 = Wa.shape[1]
    h2 = Wb.shape[1]
    bn = 400
    assert n % bn == 0
    nb = n // bn

    def body(h_ref, a0_ref, a1_ref, wa_ref, ba_ref, wb_ref, bb_ref, o_ref):
        u = h_ref[...] + a0_ref[...] + a1_ref[...]
        t = jax.nn.relu(jnp.dot(u, wa_ref[...],
                                preferred_element_type=jnp.float32) + ba_ref[...])
        v = jnp.dot(t, wb_ref[...],
                    preferred_element_type=jnp.float32) + bb_ref[...]
        o_ref[...] = jax.nn.relu(v)

    return pl.pallas_call(
        body,
        grid=(nb,),
        in_specs=[
            pl.BlockSpec((bn, d), lambda i: (i, 0)),
            pl.BlockSpec((bn, d), lambda i: (i, 0)),
            pl.BlockSpec((bn, d), lambda i: (i + nb, 0)),
            pl.BlockSpec((d, h1), lambda i: (0, 0)),
            pl.BlockSpec((1, h1), lambda i: (0, 0)),
            pl.BlockSpec((h1, h2), lambda i: (0, 0)),
            pl.BlockSpec((1, h2), lambda i: (0, 0)),
        ],
        out_specs=pl.BlockSpec((bn, h2), lambda i: (i, 0)),
        out_shape=jax.ShapeDtypeStruct((n, h2), jnp.float32),
    )(hin, agg2, agg2, Wa, ba.reshape(1, h1), Wb, bb.reshape(1, h2))


def _pool_head(h, batch, g_count, Wout, bout, Wp1, bp1, Wp2, bp2):
    n, d = h.shape
    emb = Wout.shape[1]
    out = Wp2.shape[1]
    bn = 400
    assert n % bn == 0
    nb = n // bn
    assert out <= emb

    ext = out
    wout_e = jnp.concatenate([Wout, jnp.zeros((d, ext), jnp.float32)], axis=1)
    bout_e = jnp.concatenate([bout.reshape(1, emb),
                              jnp.ones((1, ext), jnp.float32)], axis=1)
    batch3 = batch.reshape(nb, 1, bn)

    def body(h_ref, b_ref, wo_ref, bo_ref, w1_ref, b1_ref, w2_ref, b2_ref,
             o_ref, acc_ref):
        i = pl.program_id(0)

        @pl.when(i == 0)
        def _():
            acc_ref[...] = jnp.zeros_like(acc_ref)

        e = jnp.dot(h_ref[...], wo_ref[...],
                    preferred_element_type=jnp.float32) + bo_ref[...]
        brow = b_ref[0]
        seg = lax.broadcasted_iota(jnp.int32, (g_count, bn), 0)
        onehot_t = (jnp.broadcast_to(brow, (g_count, bn)) == seg)
        acc_ref[...] += jnp.dot(onehot_t.astype(jnp.float32), e,
                                preferred_element_type=jnp.float32)

        @pl.when(i == nb - 1)
        def _():
            acc = acc_ref[...]
            cnt = acc[:, emb:]
            reps = emb // ext
            denom = jnp.maximum(
                jnp.concatenate([cnt] * reps, axis=1), 1.0)
            gmean = acc[:, :emb] / denom
            t = jax.nn.relu(jnp.dot(gmean, w1_ref[...],
                                    preferred_element_type=jnp.float32)
                            + b1_ref[...])
            z = jnp.dot(t, w2_ref[...],
                        preferred_element_type=jnp.float32) + b2_ref[...]
            nrm = jnp.sqrt(jnp.sum(z * z, axis=1, keepdims=True))
            o_ref[...] = z / (nrm + 1e-8)

    return pl.pallas_call(
        body,
        grid=(nb,),
        in_specs=[
            pl.BlockSpec((bn, d), lambda i: (i, 0)),
            pl.BlockSpec((1, 1, bn), lambda i: (i, 0, 0)),
            pl.BlockSpec((d, emb + ext), lambda i: (0, 0)),
            pl.BlockSpec((1, emb + ext), lambda i: (0, 0)),
            pl.BlockSpec((emb, emb), lambda i: (0, 0)),
            pl.BlockSpec((1, emb), lambda i: (0, 0)),
            pl.BlockSpec((emb, out), lambda i: (0, 0)),
            pl.BlockSpec((1, out), lambda i: (0, 0)),
        ],
        out_specs=pl.BlockSpec((g_count, out), lambda i: (0, 0)),
        out_shape=jax.ShapeDtypeStruct((g_count, out), jnp.float32),
        scratch_shapes=[pltpu.VMEM((g_count, emb + ext), jnp.float32)],
    )(h, batch3, wout_e, bout_e, Wp1, bp1.reshape(1, emb), Wp2,
      bp2.reshape(1, out))


def kernel(x, edge_index, edge_attr, batch,
           We1, be1, W1a, b1a, W1b, b1b,
           We2, be2, W2a, b2a, W2b, b2b,
           Wout, bout, Wp1, bp1, Wp2, bp2):
    n = x.shape[0]
    g_count = 64

    e1, src, dst = _edge_embed1(edge_attr, edge_index, We1, be1)

    xc = _pallas_copy(x)
    agg1 = _edge_pass(xc, e1, src, dst)
    e2 = _edge_embed2(edge_attr, We2, be2)
    h1 = _node_update(x, agg1, W1a, b1a, W1b, b1b)

    agg2 = _edge_pass(h1, e2, src, dst)
    h2 = _node_update(h1, agg2, W2a, b2a, W2b, b2b)

    return _pool_head(h2, batch, g_count, Wout, bout, Wp1, bp1, Wp2, bp2)


# speedup vs baseline: 1.1825x; 1.1825x over previous
import functools

import jax
import jax.numpy as jnp
from jax import lax
from jax.experimental import pallas as pl
from jax.experimental.pallas import tpu as pltpu
from jax.experimental.pallas import tpu_sc as plsc

_NC = 2
_NS = 16
_NW = _NC * _NS

_CK = 80


def _edge_pass(h, e, src, dst):
    n, d = h.shape
    ecount = src.shape[0]
    assert e.shape == (ecount, d) and e.dtype == jnp.float32
    assert ecount % _NW == 0
    epw = ecount // _NW
    assert epw % _CK == 0
    nchunk = epw // _CK
    assert nchunk % 2 == 1 and nchunk >= 5
    assert n % 8 == 0
    rows_a = (n // 8 // _NS) * 8
    rows_last = n - (_NS - 1) * rows_a
    zr = _CK
    assert (rows_a % zr) % 8 == 0 and (rows_last - rows_a) % 8 == 0
    assert rows_last - rows_a <= zr

    mesh = plsc.VectorSubcoreMesh(core_axis_name="c", subcore_axis_name="s",
                                  num_cores=_NC, num_subcores=_NS)

    @functools.partial(
        pl.kernel,
        out_type=jax.ShapeDtypeStruct((2 * n, d), jnp.float32),
        mesh=mesh,
        scratch_types=[
            [pltpu.VMEM((_CK,), jnp.int32) for _ in range(2)],
            [pltpu.VMEM((_CK,), jnp.int32) for _ in range(2)],
            [pltpu.VMEM((_CK, d), jnp.float32) for _ in range(2)],
            [pltpu.VMEM((_CK, d), jnp.float32) for _ in range(2)],
            pltpu.VMEM_SHARED((n, d), jnp.float32),
            [pltpu.SemaphoreType.DMA for _ in range(2)],
            [pltpu.SemaphoreType.DMA for _ in range(2)],
        ],
    )
    def body(h_hbm, e_hbm, src_hbm, dst_hbm, out_hbm,
             sidx, didx, rows, ebuf, agg, psem, gsem):
        core = lax.axis_index("c")
        sid = lax.axis_index("s")
        wid = sid * _NC + core
        base = wid * epw

        nvec = d // 16

        @plsc.parallel_loop(0, zr, 1, unroll=4)
        def _(r):
            for j in range(nvec):
                rows[0][r, pl.ds(j * 16, 16)] = jnp.zeros((16,), jnp.float32)

        base_r = sid * rows_a
        for k in range(rows_a // zr):
            pltpu.sync_copy(rows[0].at[pl.ds(0, zr)],
                            agg.at[pl.ds(base_r + k * zr, zr)])
        rem = rows_a - (rows_a // zr) * zr
        if rem:
            pltpu.sync_copy(rows[0].at[pl.ds(0, rem)],
                            agg.at[pl.ds(base_r + rows_a - rem, rem)])
        extra = rows_last - rows_a

        @pl.when(sid == _NS - 1)
        def _():
            pltpu.sync_copy(rows[0].at[pl.ds(0, extra)],
                            agg.at[pl.ds(base_r + rows_a, extra)])

        plsc.subcore_barrier()

        def prefetch_copies(c, b):
            off = base + c * _CK
            return [
                (src_hbm.at[pl.ds(off, _CK)], sidx[b]),
                (dst_hbm.at[pl.ds(off, _CK)], didx[b]),
                (e_hbm.at[pl.ds(off, _CK)], ebuf[b]),
            ]

        def fire_prefetch(c, b):
            for s, t in prefetch_copies(c, b):
                pltpu.async_copy(s, t, psem[b])

        def wait_prefetch(c, b):
            for s, t in prefetch_copies(c, b):
                pltpu.make_async_copy(s, t, psem[b]).wait()

        def gather_copies(b):
            return [(h_hbm.at[sidx[b]], rows[b])]

        def fire_gather(b):
            for s, t in gather_copies(b):
                pltpu.async_copy(s, t, gsem[b])

        def wait_gather(b):
            for s, t in gather_copies(b):
                pltpu.make_async_copy(s, t, gsem[b]).wait()

        def compute(b):
            @plsc.parallel_loop(0, _CK, 1, unroll=4)
            def _(r):
                for j in range(nvec):
                    sl = pl.ds(j * 16, 16)
                    rows[b][r, sl] = jnp.maximum(
                        rows[b][r, sl] + ebuf[b][r, sl], 0.0)

        def scatter(b):
            pltpu.sync_copy(rows[b], agg.at[didx[b]], add=True)

        def half(c, b):
            bn = 1 - b
            wait_prefetch(c + 1, bn)
            fire_gather(bn)
            wait_gather(b)
            compute(b)
            scatter(b)
            fire_prefetch(c + 2, b)

        fire_prefetch(0, 0)
        wait_prefetch(0, 0)
        fire_gather(0)
        fire_prefetch(1, 1)

        def pair(k, _):
            c = 2 * k
            half(c, 0)
            half(c + 1, 1)
            return 0

        lax.fori_loop(0, (nchunk - 3) // 2, pair, 0)

        half(nchunk - 3, 0)
        wait_prefetch(nchunk - 1, 0)
        fire_gather(0)
        wait_gather(1)
        compute(1)
        scatter(1)
        wait_gather(0)
        compute(0)
        scatter(0)

        plsc.subcore_barrier()

        @pl.when(sid < _NS - 1)
        def _():
            pltpu.sync_copy(agg.at[pl.ds(base_r, rows_a)],
                            out_hbm.at[pl.ds(core * n + base_r, rows_a)])

        @pl.when(sid == _NS - 1)
        def _():
            pltpu.sync_copy(agg.at[pl.ds(base_r, rows_last)],
                            out_hbm.at[pl.ds(core * n + base_r, rows_last)])

    return body(h, e, src, dst)


def _edge_embed1(edge_attr_t, edge_index, We, b):
    de, ecount = edge_attr_t.shape
    d1 = We.shape[1]
    be = 3200
    assert ecount % be == 0
    grid = ecount // be

    def body(ea_ref, ei_ref, w_ref, b_ref, o_ref, src_ref, dst_ref):
        o_ref[...] = lax.dot_general(
            ea_ref[...], w_ref[...], (((0,), (0,)), ((), ())),
            preferred_element_type=jnp.float32) + b_ref[...]

        @pl.when(pl.program_id(0) == 0)
        def _():
            src_ref[...] = ei_ref[0, :]
            dst_ref[...] = ei_ref[1, :]

    return pl.pallas_call(
        body,
        grid=(grid,),
        in_specs=[
            pl.BlockSpec((de, be), lambda i: (0, i)),
            pl.BlockSpec((2, ecount), lambda i: (0, 0)),
            pl.BlockSpec((de, d1), lambda i: (0, 0)),
            pl.BlockSpec((1, d1), lambda i: (0, 0)),
        ],
        out_specs=[
            pl.BlockSpec((be, d1), lambda i: (i, 0)),
            pl.BlockSpec((ecount,), lambda i: (0,)),
            pl.BlockSpec((ecount,), lambda i: (0,)),
        ],
        out_shape=[
            jax.ShapeDtypeStruct((ecount, d1), jnp.float32),
            jax.ShapeDtypeStruct((ecount,), jnp.int32),
            jax.ShapeDtypeStruct((ecount,), jnp.int32),
        ],
    )(edge_attr_t, edge_index, We, b.reshape(1, d1))


def _edge_embed2(edge_attr_t, We, b):
    de, ecount = edge_attr_t.shape
    d1 = We.shape[1]
    be = 3200
    assert ecount % be == 0
    grid = ecount // be

    def body(ea_ref, w_ref, b_ref, o_ref):
        o_ref[...] = lax.dot_general(
            ea_ref[...], w_ref[...], (((0,), (0,)), ((), ())),
            preferred_element_type=jnp.float32) + b_ref[...]

    return pl.pallas_call(
        body,
        grid=(grid,),
        in_specs=[
            pl.BlockSpec((de, be), lambda i: (0, i)),
            pl.BlockSpec((de, d1), lambda i: (0, 0)),
            pl.BlockSpec((1, d1), lambda i: (0, 0)),
        ],
        out_specs=pl.BlockSpec((be, d1), lambda i: (i, 0)),
        out_shape=jax.ShapeDtypeStruct((ecount, d1), jnp.float32),
    )(edge_attr_t, We, b.reshape(1, d1))


def _pallas_copy(x):
    n, d = x.shape
    bn = 400
    assert n % bn == 0

    def body(i_ref, o_ref):
        o_ref[...] = i_ref[...]

    return pl.pallas_call(
        body,
        grid=(n // bn,),
        in_specs=[pl.BlockSpec((bn, d), lambda i: (i, 0))],
        out_specs=pl.BlockSpec((bn, d), lambda i: (i, 0)),
        out_shape=jax.ShapeDtypeStruct((n, d), jnp.float32),
    )(x)


def _node_update(hin, agg2, Wa, ba, Wb, bb):
    n, d = hin.shape
    h1 = Wa.shape[1]
    h2 = Wb.shape[1]
    bn = 400
    assert n % bn == 0
    nb = n // bn

    def body(h_ref, a0_ref, a1_ref, wa_ref, ba_ref, wb_ref, bb_ref, o_ref):
        u = h_ref[...] + a0_ref[...] + a1_ref[...]
        t = jax.nn.relu(jnp.dot(u, wa_ref[...],
                                preferred_element_type=jnp.float32) + ba_ref[...])
        v = jnp.dot(t, wb_ref[...],
                    preferred_element_type=jnp.float32) + bb_ref[...]
        o_ref[...] = jax.nn.relu(v)

    return pl.pallas_call(
        body,
        grid=(nb,),
        in_specs=[
            pl.BlockSpec((bn, d), lambda i: (i, 0)),
            pl.BlockSpec((bn, d), lambda i: (i, 0)),
            pl.BlockSpec((bn, d), lambda i: (i + nb, 0)),
            pl.BlockSpec((d, h1), lambda i: (0, 0)),
            pl.BlockSpec((1, h1), lambda i: (0, 0)),
            pl.BlockSpec((h1, h2), lambda i: (0, 0)),
            pl.BlockSpec((1, h2), lambda i: (0, 0)),
        ],
        out_specs=pl.BlockSpec((bn, h2), lambda i: (i, 0)),
        out_shape=jax.ShapeDtypeStruct((n, h2), jnp.float32),
    )(hin, agg2, agg2, Wa, ba.reshape(1, h1), Wb, bb.reshape(1, h2))


def _pool_head(h, batch, g_count, Wout, bout, Wp1, bp1, Wp2, bp2):
    n, d = h.shape
    emb = Wout.shape[1]
    out = Wp2.shape[1]
    bn = 400
    assert n % bn == 0
    nb = n // bn
    assert out <= emb

    ext = out
    wout_e = jnp.concatenate([Wout, jnp.zeros((d, ext), jnp.float32)], axis=1)
    bout_e = jnp.concatenate([bout.reshape(1, emb),
                              jnp.ones((1, ext), jnp.float32)], axis=1)
    batch3 = batch.reshape(nb, 1, bn)

    def body(h_ref, b_ref, wo_ref, bo_ref, w1_ref, b1_ref, w2_ref, b2_ref,
             o_ref, acc_ref):
        i = pl.program_id(0)

        @pl.when(i == 0)
        def _():
            acc_ref[...] = jnp.zeros_like(acc_ref)

        e = jnp.dot(h_ref[...], wo_ref[...],
                    preferred_element_type=jnp.float32) + bo_ref[...]
        brow = b_ref[0]
        seg = lax.broadcasted_iota(jnp.int32, (g_count, bn), 0)
        onehot_t = (jnp.broadcast_to(brow, (g_count, bn)) == seg)
        acc_ref[...] += jnp.dot(onehot_t.astype(jnp.float32), e,
                                preferred_element_type=jnp.float32)

        @pl.when(i == nb - 1)
        def _():
            acc = acc_ref[...]
            cnt = acc[:, emb:]
            reps = emb // ext
            denom = jnp.maximum(
                jnp.concatenate([cnt] * reps, axis=1), 1.0)
            gmean = acc[:, :emb] / denom
            t = jax.nn.relu(jnp.dot(gmean, w1_ref[...],
                                    preferred_element_type=jnp.float32)
                            + b1_ref[...])
            z = jnp.dot(t, w2_ref[...],
                        preferred_element_type=jnp.float32) + b2_ref[...]
            nrm = jnp.sqrt(jnp.sum(z * z, axis=1, keepdims=True))
            o_ref[...] = z / (nrm + 1e-8)

    return pl.pallas_call(
        body,
        grid=(nb,),
        in_specs=[
            pl.BlockSpec((bn, d), lambda i: (i, 0)),
            pl.BlockSpec((1, 1, bn), lambda i: (i, 0, 0)),
            pl.BlockSpec((d, emb + ext), lambda i: (0, 0)),
            pl.BlockSpec((1, emb + ext), lambda i: (0, 0)),
            pl.BlockSpec((emb, emb), lambda i: (0, 0)),
            pl.BlockSpec((1, emb), lambda i: (0, 0)),
            pl.BlockSpec((emb, out), lambda i: (0, 0)),
            pl.BlockSpec((1, out), lambda i: (0, 0)),
        ],
        out_specs=pl.BlockSpec((g_count, out), lambda i: (0, 0)),
        out_shape=jax.ShapeDtypeStruct((g_count, out), jnp.float32),
        scratch_shapes=[pltpu.VMEM((g_count, emb + ext), jnp.float32)],
    )(h, batch3, wout_e, bout_e, Wp1, bp1.reshape(1, emb), Wp2,
      bp2.reshape(1, out))


def kernel(x, edge_index, edge_attr, batch,
           We1, be1, W1a, b1a, W1b, b1b,
           We2, be2, W2a, b2a, W2b, b2b,
           Wout, bout, Wp1, bp1, Wp2, bp2):
    n = x.shape[0]
    g_count = 64

    ea_t = edge_attr.T
    e1, src, dst = _edge_embed1(ea_t, edge_index, We1, be1)

    agg1 = _edge_pass(x, e1, src, dst)
    e2 = _edge_embed2(ea_t, We2, be2)
    h1 = _node_update(x, agg1, W1a, b1a, W1b, b1b)

    agg2 = _edge_pass(h1, e2, src, dst)
    h2 = _node_update(h1, agg2, W2a, b2a, W2b, b2b)

    return _pool_head(h2, batch, g_count, Wout, bout, Wp1, bp1, Wp2, bp2)


# speedup vs baseline: 1.3517x; 1.1431x over previous
import functools

import jax
import jax.numpy as jnp
from jax import lax
from jax.experimental import pallas as pl
from jax.experimental.pallas import tpu as pltpu
from jax.experimental.pallas import tpu_sc as plsc

_NC = 2
_NS = 16
_NW = _NC * _NS

_CK = 80


def _edge_pass(h, e, src, dst):
    n, d = h.shape
    ecount = src.shape[0]
    assert e.shape == (ecount, d) and e.dtype == jnp.float32
    assert ecount % _NW == 0
    epw = ecount // _NW
    assert epw % _CK == 0
    nchunk = epw // _CK
    assert nchunk % 2 == 1 and nchunk >= 7
    assert n % 8 == 0
    rows_a = (n // 8 // _NS) * 8
    rows_last = n - (_NS - 1) * rows_a
    zr = _CK
    assert (rows_a % zr) % 8 == 0 and (rows_last - rows_a) % 8 == 0
    assert rows_last - rows_a <= zr

    mesh = plsc.VectorSubcoreMesh(core_axis_name="c", subcore_axis_name="s",
                                  num_cores=_NC, num_subcores=_NS)

    @functools.partial(
        pl.kernel,
        out_type=jax.ShapeDtypeStruct((2 * n, d), jnp.float32),
        mesh=mesh,
        scratch_types=[
            [pltpu.VMEM((_CK,), jnp.int32) for _ in range(2)],
            [pltpu.VMEM((_CK,), jnp.int32) for _ in range(2)],
            [pltpu.VMEM((_CK, d), jnp.float32) for _ in range(2)],
            [pltpu.VMEM((_CK, d), jnp.float32) for _ in range(2)],
            pltpu.VMEM_SHARED((n, d), jnp.float32),
            [pltpu.SemaphoreType.DMA for _ in range(2)],
            [pltpu.SemaphoreType.DMA for _ in range(2)],
            [pltpu.SemaphoreType.DMA for _ in range(2)],
            [pltpu.SemaphoreType.DMA for _ in range(2)],
        ],
    )
    def body(h_hbm, e_hbm, src_hbm, dst_hbm, out_hbm,
             sidx, didx, rows, ebuf, agg, psem, dsem, gsem, ssem):
        core = lax.axis_index("c")
        sid = lax.axis_index("s")
        wid = sid * _NC + core
        base = wid * epw

        nvec = d // 16

        @plsc.parallel_loop(0, zr, 1, unroll=4)
        def _(r):
            for j in range(nvec):
                rows[0][r, pl.ds(j * 16, 16)] = jnp.zeros((16,), jnp.float32)

        base_r = sid * rows_a
        for k in range(rows_a // zr):
            pltpu.sync_copy(rows[0].at[pl.ds(0, zr)],
                            agg.at[pl.ds(base_r + k * zr, zr)])
        rem = rows_a - (rows_a // zr) * zr
        if rem:
            pltpu.sync_copy(rows[0].at[pl.ds(0, rem)],
                            agg.at[pl.ds(base_r + rows_a - rem, rem)])
        extra = rows_last - rows_a

        @pl.when(sid == _NS - 1)
        def _():
            pltpu.sync_copy(rows[0].at[pl.ds(0, extra)],
                            agg.at[pl.ds(base_r + rows_a, extra)])

        plsc.subcore_barrier()

        def se_copies(c, b):
            off = base + c * _CK
            return [
                (src_hbm.at[pl.ds(off, _CK)], sidx[b]),
                (e_hbm.at[pl.ds(off, _CK)], ebuf[b]),
            ]

        def fire_pre(c, b):
            for s, t in se_copies(c, b):
                pltpu.async_copy(s, t, psem[b])

        def wait_pre(c, b):
            for s, t in se_copies(c, b):
                pltpu.make_async_copy(s, t, psem[b]).wait()

        def fire_didx(c, b):
            off = base + c * _CK
            pltpu.async_copy(dst_hbm.at[pl.ds(off, _CK)], didx[b], dsem[b])

        def wait_didx(c, b):
            off = base + c * _CK
            pltpu.make_async_copy(dst_hbm.at[pl.ds(off, _CK)], didx[b],
                                  dsem[b]).wait()

        def fire_gather(b):
            pltpu.async_copy(h_hbm.at[sidx[b]], rows[b], gsem[b])

        def wait_gather(b):
            pltpu.make_async_copy(h_hbm.at[sidx[b]], rows[b], gsem[b]).wait()

        def fire_scatter(b):
            pltpu.async_copy(rows[b], agg.at[didx[b]], ssem[b], add=True)

        def wait_scatter(b):
            pltpu.make_async_copy(rows[b], agg.at[didx[b]], ssem[b]).wait()

        def compute(b):
            @plsc.parallel_loop(0, _CK, 1, unroll=4)
            def _(r):
                for j in range(nvec):
                    sl = pl.ds(j * 16, 16)
                    rows[b][r, sl] = jnp.maximum(
                        rows[b][r, sl] + ebuf[b][r, sl], 0.0)

        fire_pre(0, 0)
        fire_didx(0, 0)
        fire_pre(1, 1)
        fire_didx(1, 1)
        wait_pre(0, 0)
        fire_gather(0)

        wait_pre(1, 1)
        fire_gather(1)
        wait_gather(0)
        compute(0)
        wait_didx(0, 0)
        fire_scatter(0)
        fire_pre(2, 0)

        def half(c, b):
            bn = 1 - b
            wait_pre(c + 1, bn)
            wait_scatter(bn)
            fire_gather(bn)
            fire_didx(c + 1, bn)
            wait_gather(b)
            compute(b)
            wait_didx(c, b)
            fire_scatter(b)
            fire_pre(c + 2, b)

        half(1, 1)

        def pair(k, _):
            c = 2 * k + 2
            half(c, 0)
            half(c + 1, 1)
            return 0

        lax.fori_loop(0, (nchunk - 5) // 2, pair, 0)

        half(nchunk - 3, 0)
        c = nchunk - 2
        wait_pre(c + 1, 0)
        wait_scatter(0)
        fire_gather(0)
        fire_didx(c + 1, 0)
        wait_gather(1)
        compute(1)
        wait_didx(c, 1)
        fire_scatter(1)
        wait_scatter(1)
        wait_gather(0)
        compute(0)
        wait_didx(c + 1, 0)
        fire_scatter(0)
        wait_scatter(0)

        plsc.subcore_barrier()

        @pl.when(sid < _NS - 1)
        def _():
            pltpu.sync_copy(agg.at[pl.ds(base_r, rows_a)],
                            out_hbm.at[pl.ds(core * n + base_r, rows_a)])

        @pl.when(sid == _NS - 1)
        def _():
            pltpu.sync_copy(agg.at[pl.ds(base_r, rows_last)],
                            out_hbm.at[pl.ds(core * n + base_r, rows_last)])

    return body(h, e, src, dst)


def _edge_embed1(edge_attr_t, edge_index, We, b):
    de, ecount = edge_attr_t.shape
    d1 = We.shape[1]
    be = 3200
    assert ecount % be == 0
    grid = ecount // be

    def body(ea_ref, ei_ref, w_ref, b_ref, o_ref, src_ref, dst_ref):
        o_ref[...] = lax.dot_general(
            ea_ref[...], w_ref[...], (((0,), (0,)), ((), ())),
            preferred_element_type=jnp.float32) + b_ref[...]

        @pl.when(pl.program_id(0) == 0)
        def _():
            src_ref[...] = ei_ref[0, :]
            dst_ref[...] = ei_ref[1, :]

    return pl.pallas_call(
        body,
        grid=(grid,),
        in_specs=[
            pl.BlockSpec((de, be), lambda i: (0, i)),
            pl.BlockSpec((2, ecount), lambda i: (0, 0)),
            pl.BlockSpec((de, d1), lambda i: (0, 0)),
            pl.BlockSpec((1, d1), lambda i: (0, 0)),
        ],
        out_specs=[
            pl.BlockSpec((be, d1), lambda i: (i, 0)),
            pl.BlockSpec((ecount,), lambda i: (0,)),
            pl.BlockSpec((ecount,), lambda i: (0,)),
        ],
        out_shape=[
            jax.ShapeDtypeStruct((ecount, d1), jnp.float32),
            jax.ShapeDtypeStruct((ecount,), jnp.int32),
            jax.ShapeDtypeStruct((ecount,), jnp.int32),
        ],
    )(edge_attr_t, edge_index, We, b.reshape(1, d1))


def _edge_embed2(edge_attr_t, We, b):
    de, ecount = edge_attr_t.shape
    d1 = We.shape[1]
    be = 3200
    assert ecount % be == 0
    grid = ecount // be

    def body(ea_ref, w_ref, b_ref, o_ref):
        o_ref[...] = lax.dot_general(
            ea_ref[...], w_ref[...], (((0,), (0,)), ((), ())),
            preferred_element_type=jnp.float32) + b_ref[...]

    return pl.pallas_call(
        body,
        grid=(grid,),
        in_specs=[
            pl.BlockSpec((de, be), lambda i: (0, i)),
            pl.BlockSpec((de, d1), lambda i: (0, 0)),
            pl.BlockSpec((1, d1), lambda i: (0, 0)),
        ],
        out_specs=pl.BlockSpec((be, d1), lambda i: (i, 0)),
        out_shape=jax.ShapeDtypeStruct((ecount, d1), jnp.float32),
    )(edge_attr_t, We, b.reshape(1, d1))


def _pallas_copy(x):
    n, d = x.shape
    bn = 400
    assert n % bn == 0

    def body(i_ref, o_ref):
        o_ref[...] = i_ref[...]

    return pl.pallas_call(
        body,
        grid=(n // bn,),
        in_specs=[pl.BlockSpec((bn, d), lambda i: (i, 0))],
        out_specs=pl.BlockSpec((bn, d), lambda i: (i, 0)),
        out_shape=jax.ShapeDtypeStruct((n, d), jnp.float32),
    )(x)


def _node_update(hin, agg2, Wa, ba, Wb, bb):
    n, d = hin.shape
    h1 = Wa.shape[1]
    h2 = Wb.shape[1]
    bn = 400
    assert n % bn == 0
    nb = n // bn

    def body(h_ref, a0_ref, a1_ref, wa_ref, ba_ref, wb_ref, bb_ref, o_ref):
        u = h_ref[...] + a0_ref[...] + a1_ref[...]
        t = jax.nn.relu(jnp.dot(u, wa_ref[...],
                                preferred_element_type=jnp.float32) + ba_ref[...])
        v = jnp.dot(t, wb_ref[...],
                    preferred_element_type=jnp.float32) + bb_ref[...]
        o_ref[...] = jax.nn.relu(v)

    return pl.pallas_call(
        body,
        grid=(nb,),
        in_specs=[
            pl.BlockSpec((bn, d), lambda i: (i, 0)),
            pl.BlockSpec((bn, d), lambda i: (i, 0)),
            pl.BlockSpec((bn, d), lambda i: (i + nb, 0)),
            pl.BlockSpec((d, h1), lambda i: (0, 0)),
            pl.BlockSpec((1, h1), lambda i: (0, 0)),
            pl.BlockSpec((h1, h2), lambda i: (0, 0)),
            pl.BlockSpec((1, h2), lambda i: (0, 0)),
        ],
        out_specs=pl.BlockSpec((bn, h2), lambda i: (i, 0)),
        out_shape=jax.ShapeDtypeStruct((n, h2), jnp.float32),
    )(hin, agg2, agg2, Wa, ba.reshape(1, h1), Wb, bb.reshape(1, h2))


def _pool_head(h, batch, g_count, Wout, bout, Wp1, bp1, Wp2, bp2):
    n, d = h.shape
    emb = Wout.shape[1]
    out = Wp2.shape[1]
    bn = 400
    assert n % bn == 0
    nb = n // bn
    assert out <= emb

    ext = out
    wout_e = jnp.concatenate([Wout, jnp.zeros((d, ext), jnp.float32)], axis=1)
    bout_e = jnp.concatenate([bout.reshape(1, emb),
                              jnp.ones((1, ext), jnp.float32)], axis=1)
    batch3 = batch.reshape(nb, 1, bn)

    def body(h_ref, b_ref, wo_ref, bo_ref, w1_ref, b1_ref, w2_ref, b2_ref,
             o_ref, acc_ref):
        i = pl.program_id(0)

        @pl.when(i == 0)
        def _():
            acc_ref[...] = jnp.zeros_like(acc_ref)

        e = jnp.dot(h_ref[...], wo_ref[...],
                    preferred_element_type=jnp.float32) + bo_ref[...]
        brow = b_ref[0]
        seg = lax.broadcasted_iota(jnp.int32, (g_count, bn), 0)
        onehot_t = (jnp.broadcast_to(brow, (g_count, bn)) == seg)
        acc_ref[...] += jnp.dot(onehot_t.astype(jnp.float32), e,
                                preferred_element_type=jnp.float32)

        @pl.when(i == nb - 1)
        def _():
            acc = acc_ref[...]
            cnt = acc[:, emb:]
            reps = emb // ext
            denom = jnp.maximum(
                jnp.concatenate([cnt] * reps, axis=1), 1.0)
            gmean = acc[:, :emb] / denom
            t = jax.nn.relu(jnp.dot(gmean, w1_ref[...],
                                    preferred_element_type=jnp.float32)
                            + b1_ref[...])
            z = jnp.dot(t, w2_ref[...],
                        preferred_element_type=jnp.float32) + b2_ref[...]
            nrm = jnp.sqrt(jnp.sum(z * z, axis=1, keepdims=True))
            o_ref[...] = z / (nrm + 1e-8)

    return pl.pallas_call(
        body,
        grid=(nb,),
        in_specs=[
            pl.BlockSpec((bn, d), lambda i: (i, 0)),
            pl.BlockSpec((1, 1, bn), lambda i: (i, 0, 0)),
            pl.BlockSpec((d, emb + ext), lambda i: (0, 0)),
            pl.BlockSpec((1, emb + ext), lambda i: (0, 0)),
            pl.BlockSpec((emb, emb), lambda i: (0, 0)),
            pl.BlockSpec((1, emb), lambda i: (0, 0)),
            pl.BlockSpec((emb, out), lambda i: (0, 0)),
            pl.BlockSpec((1, out), lambda i: (0, 0)),
        ],
        out_specs=pl.BlockSpec((g_count, out), lambda i: (0, 0)),
        out_shape=jax.ShapeDtypeStruct((g_count, out), jnp.float32),
        scratch_shapes=[pltpu.VMEM((g_count, emb + ext), jnp.float32)],
    )(h, batch3, wout_e, bout_e, Wp1, bp1.reshape(1, emb), Wp2,
      bp2.reshape(1, out))


def kernel(x, edge_index, edge_attr, batch,
           We1, be1, W1a, b1a, W1b, b1b,
           We2, be2, W2a, b2a, W2b, b2b,
           Wout, bout, Wp1, bp1, Wp2, bp2):
    n = x.shape[0]
    g_count = 64

    ea_t = edge_attr.T
    e1, src, dst = _edge_embed1(ea_t, edge_index, We1, be1)

    agg1 = _edge_pass(x, e1, src, dst)
    e2 = _edge_embed2(ea_t, We2, be2)
    h1 = _node_update(x, agg1, W1a, b1a, W1b, b1b)

    agg2 = _edge_pass(h1, e2, src, dst)
    h2 = _node_update(h1, agg2, W2a, b2a, W2b, b2b)

    return _pool_head(h2, batch, g_count, Wout, bout, Wp1, bp1, Wp2, bp2)
